# baseline (speedup 1.0000x reference)
import jax
import jax.numpy as jnp
from jax import lax
from jax.experimental import pallas as pl
from jax.experimental.pallas import tpu as pltpu
from jax.experimental.pallas import tpu_sc as plsc

M = 1000000
B = 16384
ALPHA = 0.9

NC = 2
NS = 16
NW = NC * NS
L = 16

CH = 31256
CH_LAST = M - (NW - 1) * CH
assert CH % 8 == 0 and CH_LAST % 8 == 0 and CH_LAST <= CH
NB = B // L
UNROLL_A = 8
TILE = 768
NT = TILE // 128
CAP = B + TILE + L


def _ema_body(data_hbm, idx_hbm, par_hbm, upd_hbm, out_hbm,
              och, idxv, gidx, cpos, pbuf, ubuf, dbuf, semc, semg):
    wid = lax.axis_index("s") * NC + lax.axis_index("c")
    lo = wid * CH
    is_last = wid == NW - 1

    def och_copy(n):
        return pltpu.make_async_copy(par_hbm.at[pl.ds(lo, n)],
                                     och.at[pl.ds(0, n)], semc)

    @pl.when(jnp.logical_not(is_last))
    def _():
        och_copy(CH).start()

    @pl.when(is_last)
    def _():
        och_copy(CH_LAST).start()

    pltpu.sync_copy(idx_hbm, idxv)

    size_u = (jnp.where(is_last, CH_LAST, CH)).astype(jnp.uint32)
    lane = lax.iota(jnp.int32, L)

    def stepA(i, cursor):
        base = i * UNROLL_A * L
        for k in range(UNROLL_A):
            off = base + k * L
            idx = idxv[pl.ds(off, L)]
            m = (idx - lo).astype(jnp.uint32) < size_u
            plsc.store_compressed(gidx.at[pl.ds(cursor, L)], idx, mask=m)
            plsc.store_compressed(cpos.at[pl.ds(cursor, L)],
                                  lane + off, mask=m)
            cursor = cursor + plsc.all_reduce_population_count(m)[0]
        return cursor

    n = lax.fori_loop(0, NB // UNROLL_A, stepA, jnp.int32(0))

    pad_idx = jnp.full((L,), 0, jnp.int32) + lo
    pad_pos = lane + wid * L
    for k in range(TILE // L):
        gidx[pl.ds(n + k * L, L)] = pad_idx
        cpos[pl.ds(n + k * L, L)] = pad_pos

    @pl.when(jnp.logical_not(is_last))
    def _():
        och_copy(CH).wait()

    @pl.when(is_last)
    def _():
        och_copy(CH_LAST).wait()

    def gather_copies(base):
        cps = []
        for c in range(NT):
            co = base + c * 128
            dst = pl.ds(c * 128, 128)
            cps.append(pltpu.make_async_copy(
                par_hbm.at[gidx.at[pl.ds(co, 128)]], pbuf.at[dst], semg))
            cps.append(pltpu.make_async_copy(
                upd_hbm.at[gidx.at[pl.ds(co, 128)]], ubuf.at[dst], semg))
            cps.append(pltpu.make_async_copy(
                data_hbm.at[cpos.at[pl.ds(co, 128)]], dbuf.at[dst], semg))
        return cps

    def gloop(g, _):
        base = g * TILE
        for c in gather_copies(base):
            c.start()
        for c in gather_copies(base):
            c.wait()
        for v in range(TILE // L):
            vo = v * L
            mB = (lane + (base + vo)) < n
            loc = gidx[pl.ds(base + vo, L)] - lo
            p = pbuf[pl.ds(vo, L)]
            u = ubuf[pl.ds(vo, L)]
            d = dbuf[pl.ds(vo, L)]
            nv = ALPHA * p + (1.0 - ALPHA * u) * d
            plsc.store_scatter(och, [loc], nv, mask=mB)
        return _

    lax.fori_loop(0, (n + TILE - 1) // TILE, gloop, None)

    @pl.when(jnp.logical_not(is_last))
    def _():
        pltpu.sync_copy(och, out_hbm.at[pl.ds(lo, CH)])

    @pl.when(is_last)
    def _():
        pltpu.sync_copy(och.at[pl.ds(0, CH_LAST)], out_hbm.at[pl.ds(lo, CH_LAST)])


@jax.jit
def _ema_update(data, index, parameter, updated):
    mesh = plsc.VectorSubcoreMesh(core_axis_name="c", subcore_axis_name="s",
                                  num_cores=NC, num_subcores=NS)
    return pl.kernel(
        _ema_body,
        out_type=jax.ShapeDtypeStruct((M,), jnp.float32),
        mesh=mesh,
        compiler_params=pltpu.CompilerParams(needs_layout_passes=False),
        scratch_types=[
            pltpu.VMEM((CH,), jnp.float32),
            pltpu.VMEM((B,), jnp.int32),
            pltpu.VMEM((CAP,), jnp.int32),
            pltpu.VMEM((CAP,), jnp.int32),
            pltpu.VMEM((TILE,), jnp.float32),
            pltpu.VMEM((TILE,), jnp.float32),
            pltpu.VMEM((TILE,), jnp.float32),
            pltpu.SemaphoreType.DMA,
            pltpu.SemaphoreType.DMA,
        ],
    )(data, index, parameter, updated)


def kernel(data, index, parameter, updated):
    return _ema_update(data, index, parameter, updated)


# speedup vs baseline: 1.2639x; 1.2639x over previous
import jax
import jax.numpy as jnp
from jax import lax
from jax.experimental import pallas as pl
from jax.experimental.pallas import tpu as pltpu
from jax.experimental.pallas import tpu_sc as plsc

M = 1000000
B = 16384
ALPHA = 0.9

NC = 2
NS = 16
NW = NC * NS
L = 16

CH = 31256
CH_LAST = M - (NW - 1) * CH
assert CH % 8 == 0 and CH_LAST % 8 == 0 and CH_LAST <= CH
NB = B // L
NSTAG = 8
SEG = B // NSTAG
PCAP = B + L
UNROLL_A = 8


def _ema_body(data_hbm, idx_hbm, par_hbm, upd_hbm, out_hbm,
              pch, uch, idxv, datav, gidx, dcomp, sem):
    wid = lax.axis_index("s") * NC + lax.axis_index("c")
    lo = wid * CH
    is_last = wid == NW - 1

    def batch_copies():
        cps = []
        for j in range(NSTAG):
            part = lax.rem(wid + j, NSTAG)
            off = part * SEG
            cps.append(pltpu.make_async_copy(
                idx_hbm.at[pl.ds(off, SEG)], idxv.at[pl.ds(off, SEG)], sem))
            cps.append(pltpu.make_async_copy(
                data_hbm.at[pl.ds(off, SEG)], datav.at[pl.ds(off, SEG)], sem))
        return cps

    def chunk_copies(n):
        src = par_hbm.at[pl.ds(lo, n)]
        return [
            pltpu.make_async_copy(src, pch.at[pl.ds(0, n)], sem),
            pltpu.make_async_copy(upd_hbm.at[pl.ds(lo, n)],
                                  uch.at[pl.ds(0, n)], sem),
        ]

    @pl.when(jnp.logical_not(is_last))
    def _():
        for c in chunk_copies(CH):
            c.start()

    @pl.when(is_last)
    def _():
        for c in chunk_copies(CH_LAST):
            c.start()

    for c in batch_copies():
        c.start()
    for c in batch_copies():
        c.wait()

    @pl.when(jnp.logical_not(is_last))
    def _():
        for c in chunk_copies(CH):
            c.wait()

    @pl.when(is_last)
    def _():
        for c in chunk_copies(CH_LAST):
            c.wait()

    size_u = (jnp.where(is_last, CH_LAST, CH)).astype(jnp.uint32)
    lane = lax.iota(jnp.int32, L)

    def stepA(i, cursor):
        base = i * UNROLL_A * L
        for k in range(UNROLL_A):
            off = base + k * L
            idx = idxv[pl.ds(off, L)]
            d = datav[pl.ds(off, L)]
            m = (idx - lo).astype(jnp.uint32) < size_u
            plsc.store_compressed(gidx.at[pl.ds(cursor, L)], idx, mask=m)
            plsc.store_compressed(dcomp.at[pl.ds(cursor, L)], d, mask=m)
            cursor = cursor + plsc.all_reduce_population_count(m)[0]
        return cursor

    n = lax.fori_loop(0, NB // UNROLL_A, stepA, jnp.int32(0))
    nv_regs = (n + (L - 1)) // L

    def gather_step(v, _):
        vo = v * L
        mB = (lane + vo) < n
        loc = gidx[pl.ds(vo, L)] - lo
        p = plsc.load_gather(pch, [loc], mask=mB)
        u = plsc.load_gather(uch, [loc], mask=mB)
        d = dcomp[pl.ds(vo, L)]
        dcomp[pl.ds(vo, L)] = ALPHA * p + (1.0 - ALPHA * u) * d
        return _

    lax.fori_loop(0, nv_regs, gather_step, None)

    def scatter_step(v, _):
        vo = v * L
        mB = (lane + vo) < n
        loc = gidx[pl.ds(vo, L)] - lo
        plsc.store_scatter(pch, [loc], dcomp[pl.ds(vo, L)], mask=mB)
        return _

    lax.fori_loop(0, nv_regs, scatter_step, None)

    @pl.when(jnp.logical_not(is_last))
    def _():
        pltpu.sync_copy(pch, out_hbm.at[pl.ds(lo, CH)])

    @pl.when(is_last)
    def _():
        pltpu.sync_copy(pch.at[pl.ds(0, CH_LAST)], out_hbm.at[pl.ds(lo, CH_LAST)])


@jax.jit
def _ema_update(data, index, parameter, updated):
    mesh = plsc.VectorSubcoreMesh(core_axis_name="c", subcore_axis_name="s",
                                  num_cores=NC, num_subcores=NS)
    return pl.kernel(
        _ema_body,
        out_type=jax.ShapeDtypeStruct((M,), jnp.float32),
        mesh=mesh,
        compiler_params=pltpu.CompilerParams(needs_layout_passes=False),
        scratch_types=[
            pltpu.VMEM((CH,), jnp.float32),
            pltpu.VMEM((CH,), jnp.float32),
            pltpu.VMEM((B,), jnp.int32),
            pltpu.VMEM((B,), jnp.float32),
            pltpu.VMEM((PCAP,), jnp.int32),
            pltpu.VMEM((PCAP,), jnp.float32),
            pltpu.SemaphoreType.DMA,
        ],
    )(data, index, parameter, updated)


def kernel(data, index, parameter, updated):
    return _ema_update(data, index, parameter, updated)


# speedup vs baseline: 1.2796x; 1.0124x over previous
import jax
import jax.numpy as jnp
from jax import lax
from jax.experimental import pallas as pl
from jax.experimental.pallas import tpu as pltpu
from jax.experimental.pallas import tpu_sc as plsc

M = 1000000
B = 16384
ALPHA = 0.9

NC = 2
NS = 16
NW = NC * NS
L = 16

CH = 31256
CH_LAST = M - (NW - 1) * CH
assert CH % 8 == 0 and CH_LAST % 8 == 0 and CH_LAST <= CH
NB = B // L
NSTAG = 8
SEG = B // NSTAG
PCAP = B + L
UNROLL_A = 8


def _ema_body(data_hbm, idx_hbm, par_hbm, upd_hbm, out_hbm,
              pch, uch, idxv, datav, gidx, dcomp, semb, semc):
    wid = lax.axis_index("s") * NC + lax.axis_index("c")
    lo = wid * CH
    is_last = wid == NW - 1

    def batch_copies():
        cps = []
        for j in range(NSTAG):
            part = lax.rem(wid + j, NSTAG)
            off = part * SEG
            cps.append(pltpu.make_async_copy(
                idx_hbm.at[pl.ds(off, SEG)], idxv.at[pl.ds(off, SEG)], semb))
            cps.append(pltpu.make_async_copy(
                data_hbm.at[pl.ds(off, SEG)], datav.at[pl.ds(off, SEG)], semb))
        return cps

    def chunk_copies(n):
        src = par_hbm.at[pl.ds(lo, n)]
        return [
            pltpu.make_async_copy(src, pch.at[pl.ds(0, n)], semc),
            pltpu.make_async_copy(upd_hbm.at[pl.ds(lo, n)],
                                  uch.at[pl.ds(0, n)], semc),
        ]

    @pl.when(jnp.logical_not(is_last))
    def _():
        for c in chunk_copies(CH):
            c.start()

    @pl.when(is_last)
    def _():
        for c in chunk_copies(CH_LAST):
            c.start()

    for c in batch_copies():
        c.start()
    for c in batch_copies():
        c.wait()

    size_u = (jnp.where(is_last, CH_LAST, CH)).astype(jnp.uint32)
    lane = lax.iota(jnp.int32, L)

    def stepA(i, cursor):
        base = i * UNROLL_A * L
        for k in range(UNROLL_A):
            off = base + k * L
            idx = idxv[pl.ds(off, L)]
            d = datav[pl.ds(off, L)]
            m = (idx - lo).astype(jnp.uint32) < size_u
            plsc.store_compressed(gidx.at[pl.ds(cursor, L)], idx, mask=m)
            plsc.store_compressed(dcomp.at[pl.ds(cursor, L)], d, mask=m)
            cursor = cursor + plsc.all_reduce_population_count(m)[0]
        return cursor

    n = lax.fori_loop(0, NB // UNROLL_A, stepA, jnp.int32(0))
    nv_regs = (n + (L - 1)) // L

    @pl.when(jnp.logical_not(is_last))
    def _():
        for c in chunk_copies(CH):
            c.wait()

    @pl.when(is_last)
    def _():
        for c in chunk_copies(CH_LAST):
            c.wait()


    def gather_step(v, _):
        vo = v * L
        mB = (lane + vo) < n
        loc = gidx[pl.ds(vo, L)] - lo
        p = plsc.load_gather(pch, [loc], mask=mB)
        u = plsc.load_gather(uch, [loc], mask=mB)
        d = dcomp[pl.ds(vo, L)]
        dcomp[pl.ds(vo, L)] = ALPHA * p + (1.0 - ALPHA * u) * d
        return _

    lax.fori_loop(0, nv_regs, gather_step, None)

    def scatter_step(v, _):
        vo = v * L
        mB = (lane + vo) < n
        loc = gidx[pl.ds(vo, L)] - lo
        plsc.store_scatter(pch, [loc], dcomp[pl.ds(vo, L)], mask=mB)
        return _

    lax.fori_loop(0, nv_regs, scatter_step, None)

    @pl.when(jnp.logical_not(is_last))
    def _():
        pltpu.sync_copy(pch, out_hbm.at[pl.ds(lo, CH)])

    @pl.when(is_last)
    def _():
        pltpu.sync_copy(pch.at[pl.ds(0, CH_LAST)], out_hbm.at[pl.ds(lo, CH_LAST)])


@jax.jit
def _ema_update(data, index, parameter, updated):
    mesh = plsc.VectorSubcoreMesh(core_axis_name="c", subcore_axis_name="s",
                                  num_cores=NC, num_subcores=NS)
    return pl.kernel(
        _ema_body,
        out_type=jax.ShapeDtypeStruct((M,), jnp.float32),
        mesh=mesh,
        compiler_params=pltpu.CompilerParams(needs_layout_passes=False),
        scratch_types=[
            pltpu.VMEM((CH,), jnp.float32),
            pltpu.VMEM((CH,), jnp.float32),
            pltpu.VMEM((B,), jnp.int32),
            pltpu.VMEM((B,), jnp.float32),
            pltpu.VMEM((PCAP,), jnp.int32),
            pltpu.VMEM((PCAP,), jnp.float32),
            pltpu.SemaphoreType.DMA,
            pltpu.SemaphoreType.DMA,
        ],
    )(data, index, parameter, updated)


def kernel(data, index, parameter, updated):
    return _ema_update(data, index, parameter, updated)
